# Initial kernel scaffold; baseline (speedup 1.0000x reference)
import jax
import jax.numpy as jnp
from jax.experimental import pallas as pl


def kernel(x, edge_index, tar_ei, beta, xcn_w1, xcn_b1, xcn_w2, xcn_b2, xcn_w3, xcn_b3, xij_w1, xij_b1, xij_w2, xij_b2, lin_w1, lin_b1, lin_w2, lin_b2):
    raise NotImplementedError("write your pallas kernel here")


# speedup vs baseline: 2.4526x; 2.4526x over previous
import functools

import jax
import jax.numpy as jnp
from jax import lax
from jax.experimental import pallas as pl

N = 10000
B = 65536
IN_CH = 128
HID = 256
TBL = 128
BLK = 512


def _table_body(w1, b1, w2, b2, w3, b3, beta, out_ref):
    c = lax.broadcasted_iota(jnp.int32, (TBL, 1), 0).astype(jnp.float32)
    h = jax.nn.relu(c * w1[...] + b1[...])
    h = jax.nn.relu(
        jax.lax.dot_general(h, w2[...], (((1,), (0,)), ((), ())),
                            preferred_element_type=jnp.float32) + b2[...])
    t = jax.lax.dot_general(h, w3[...], (((1,), (0,)), ((), ())),
                            preferred_element_type=jnp.float32) + b3[...]
    out_ref[...] = t * beta[...]


def _main_body(cnt_ref, xij_ref, table_ref, w1, b1, w2, b2, l1, lb1, l2, lb2,
               out_ref):
    xij = xij_ref[...]
    h = jax.nn.relu(
        jax.lax.dot_general(xij, w1[...], (((1,), (0,)), ((), ())),
                            preferred_element_type=jnp.float32) + b1[...])
    hij = jax.lax.dot_general(h, w2[...], (((1,), (0,)), ((), ())),
                              preferred_element_type=jnp.float32) + b2[...]
    c = cnt_ref[...]
    onehot = (c == lax.broadcasted_iota(jnp.int32, (BLK, TBL), 1)
              ).astype(jnp.float32)
    hcn = jax.lax.dot_general(onehot, table_ref[...], (((1,), (0,)), ((), ())),
                              preferred_element_type=jnp.float32)
    z = hcn + hij
    h2 = jax.nn.relu(
        jax.lax.dot_general(z, l1[...], (((1,), (0,)), ((), ())),
                            preferred_element_type=jnp.float32) + lb1[...])
    out_ref[...] = jax.lax.dot_general(
        h2, l2[...], (((1,), (0,)), ((), ())),
        preferred_element_type=jnp.float32) + lb2[...]


def _counts_xla(edge_index, tar_ei):
    src, dst = edge_index[0], edge_index[1]
    keyv = src * N + dst
    k = jnp.sort(keyv)
    uniq = jnp.concatenate([jnp.ones((1,), bool), k[1:] != k[:-1]])
    s = k // N
    d = k % N
    W = 320
    word = s * W + (d >> 5)
    bit = (jnp.uint32(1) << (d & 31).astype(jnp.uint32))
    vals = jnp.where(uniq, bit, jnp.uint32(0))
    bm = jax.ops.segment_sum(vals, word, num_segments=N * W).reshape(N, W)
    a = bm[tar_ei[0]]
    b = bm[tar_ei[1]]
    x = a & b
    x = x - ((x >> 1) & jnp.uint32(0x55555555))
    x = (x & jnp.uint32(0x33333333)) + ((x >> 2) & jnp.uint32(0x33333333))
    x = (x + (x >> 4)) & jnp.uint32(0x0F0F0F0F)
    x = (x * jnp.uint32(0x01010101)) >> 24
    return jnp.sum(x, axis=1, dtype=jnp.uint32).astype(jnp.int32)


def kernel(x, edge_index, tar_ei, beta, xcn_w1, xcn_b1, xcn_w2, xcn_b2,
           xcn_w3, xcn_b3, xij_w1, xij_b1, xij_w2, xij_b2,
           lin_w1, lin_b1, lin_w2, lin_b2):
    counts = _counts_xla(edge_index, tar_ei)
    xij = x[tar_ei[0]] * x[tar_ei[1]]

    full = lambda shape: pl.BlockSpec(shape, lambda *_: (0,) * len(shape))
    table = pl.pallas_call(
        _table_body,
        out_shape=jax.ShapeDtypeStruct((TBL, HID), jnp.float32),
        in_specs=[full((1, HID)), full((1, HID)), full((HID, HID)),
                  full((1, HID)), full((HID, HID)), full((1, HID)),
                  full((1, 1))],
        out_specs=full((TBL, HID)),
    )(xcn_w1, xcn_b1.reshape(1, HID), xcn_w2, xcn_b2.reshape(1, HID),
      xcn_w3, xcn_b3.reshape(1, HID), beta.reshape(1, 1))

    nb = B // BLK
    out = pl.pallas_call(
        _main_body,
        grid=(nb,),
        out_shape=jax.ShapeDtypeStruct((B, 1), jnp.float32),
        in_specs=[
            pl.BlockSpec((BLK, 1), lambda i: (i, 0)),
            pl.BlockSpec((BLK, IN_CH), lambda i: (i, 0)),
            full((TBL, HID)),
            full((IN_CH, HID)), full((1, HID)),
            full((HID, HID)), full((1, HID)),
            full((HID, HID)), full((1, HID)),
            full((HID, 1)), full((1, 1)),
        ],
        out_specs=pl.BlockSpec((BLK, 1), lambda i: (i, 0)),
    )(counts.reshape(B, 1), xij, table,
      xij_w1, xij_b1.reshape(1, HID), xij_w2, xij_b2.reshape(1, HID),
      lin_w1, lin_b1.reshape(1, HID), lin_w2, lin_b2.reshape(1, 1))
    return out


# speedup vs baseline: 3.0890x; 1.2595x over previous
import functools

import jax
import jax.numpy as jnp
from jax import lax
from jax.experimental import pallas as pl
from jax.experimental.pallas import tpu as pltpu
from jax.experimental.pallas import tpu_sc as plsc

N = 10000
B = 65536
IN_CH = 128
HID = 256
TBL = 128
BLK = 512

WRDS = 320
NWK = 32
PW = B // NWK
CH = 64
NCH = PW // CH


def _pair_body(bm_hbm, tar0_hbm, tar1_hbm, x_hbm, counts_hbm, xij_hbm,
               idx_i, idx_j, rows_i, rows_j, xi, xj, xij_buf, cnt_buf, sem):
    wid = lax.axis_index("s") * 2 + lax.axis_index("c")
    base = wid * PW

    def chunk(ch, carry):
        off = base + ch * CH
        pltpu.sync_copy(tar0_hbm.at[pl.ds(off, CH)], idx_i)
        pltpu.sync_copy(tar1_hbm.at[pl.ds(off, CH)], idx_j)
        h1 = pltpu.async_copy(bm_hbm.at[idx_i], rows_i, sem)
        h2 = pltpu.async_copy(bm_hbm.at[idx_j], rows_j, sem)
        h3 = pltpu.async_copy(x_hbm.at[idx_i], xi, sem)
        h4 = pltpu.async_copy(x_hbm.at[idx_j], xj, sem)
        h1.wait()
        h2.wait()
        h3.wait()
        h4.wait()

        def pair(p, c2):
            acc = jnp.zeros((16,), jnp.int32)
            for k in range(WRDS // 16):
                v = (rows_i[p, pl.ds(k * 16, 16)]
                     & rows_j[p, pl.ds(k * 16, 16)])
                v = v - (lax.shift_right_logical(v, 1) & 0x55555555)
                v = ((v & 0x33333333)
                     + (lax.shift_right_logical(v, 2) & 0x33333333))
                v = (v + lax.shift_right_logical(v, 4)) & 0x0F0F0F0F
                acc = acc + v
            cnt_buf[p, :] = lax.shift_right_logical(
                acc * jnp.int32(0x01010101), 24)
            for k in range(IN_CH // 16):
                s = pl.ds(k * 16, 16)
                xij_buf[p, s] = xi[p, s] * xj[p, s]
            return c2
        lax.fori_loop(0, CH, pair, 0)
        pltpu.sync_copy(xij_buf, xij_hbm.at[pl.ds(off, CH)])
        pltpu.sync_copy(cnt_buf, counts_hbm.at[pl.ds(off, CH)])
        return carry

    lax.fori_loop(0, NCH, chunk, 0)


def _sc_pairs(bm, tar0, tar1, x):
    mesh = plsc.VectorSubcoreMesh(core_axis_name="c", subcore_axis_name="s")
    f = pl.kernel(
        _pair_body,
        out_type=(jax.ShapeDtypeStruct((B, 16), jnp.int32),
                  jax.ShapeDtypeStruct((B, IN_CH), jnp.float32)),
        mesh=mesh,
        scratch_types=[
            pltpu.VMEM((CH,), jnp.int32),
            pltpu.VMEM((CH,), jnp.int32),
            pltpu.VMEM((CH, WRDS), jnp.int32),
            pltpu.VMEM((CH, WRDS), jnp.int32),
            pltpu.VMEM((CH, IN_CH), jnp.float32),
            pltpu.VMEM((CH, IN_CH), jnp.float32),
            pltpu.VMEM((CH, IN_CH), jnp.float32),
            pltpu.VMEM((CH, 16), jnp.int32),
            pltpu.SemaphoreType.DMA,
        ],
        compiler_params=pltpu.CompilerParams(use_tc_tiling_on_sc=False),
    )
    return f(bm, tar0, tar1, x)


def _table_body(w1, b1, w2, b2, w3, b3, beta, out_ref):
    c = lax.broadcasted_iota(jnp.int32, (TBL, 1), 0).astype(jnp.float32)
    h = jax.nn.relu(c * w1[...] + b1[...])
    h = jax.nn.relu(
        jax.lax.dot_general(h, w2[...], (((1,), (0,)), ((), ())),
                            preferred_element_type=jnp.float32) + b2[...])
    t = jax.lax.dot_general(h, w3[...], (((1,), (0,)), ((), ())),
                            preferred_element_type=jnp.float32) + b3[...]
    out_ref[...] = t * beta[...]


def _main_body(cnt_ref, xij_ref, table_ref, w1, b1, w2, b2, l1, lb1, l2, lb2,
               out_ref):
    xij = xij_ref[...]
    h = jax.nn.relu(
        jax.lax.dot_general(xij, w1[...], (((1,), (0,)), ((), ())),
                            preferred_element_type=jnp.float32) + b1[...])
    hij = jax.lax.dot_general(h, w2[...], (((1,), (0,)), ((), ())),
                              preferred_element_type=jnp.float32) + b2[...]
    c = jnp.sum(cnt_ref[...], axis=1, keepdims=True)
    onehot = (c == lax.broadcasted_iota(jnp.int32, (BLK, TBL), 1)
              ).astype(jnp.float32)
    hcn = jax.lax.dot_general(onehot, table_ref[...], (((1,), (0,)), ((), ())),
                              preferred_element_type=jnp.float32)
    z = hcn + hij
    h2 = jax.nn.relu(
        jax.lax.dot_general(z, l1[...], (((1,), (0,)), ((), ())),
                            preferred_element_type=jnp.float32) + lb1[...])
    out_ref[...] = jax.lax.dot_general(
        h2, l2[...], (((1,), (0,)), ((), ())),
        preferred_element_type=jnp.float32) + lb2[...]


def _bitmap_xla(edge_index):
    src, dst = edge_index[0], edge_index[1]
    k = jnp.sort(src * N + dst)
    uniq = jnp.concatenate([jnp.ones((1,), bool), k[1:] != k[:-1]])
    s = k // N
    d = k % N
    word = s * WRDS + (d >> 5)
    bit = (jnp.uint32(1) << (d & 31).astype(jnp.uint32))
    vals = jnp.where(uniq, bit, jnp.uint32(0))
    bm = jax.ops.segment_sum(vals, word, num_segments=N * WRDS)
    return lax.bitcast_convert_type(bm, jnp.int32).reshape(N, WRDS)


def kernel(x, edge_index, tar_ei, beta, xcn_w1, xcn_b1, xcn_w2, xcn_b2,
           xcn_w3, xcn_b3, xij_w1, xij_b1, xij_w2, xij_b2,
           lin_w1, lin_b1, lin_w2, lin_b2):
    bm = _bitmap_xla(edge_index)
    counts, xij = _sc_pairs(bm, tar_ei[0], tar_ei[1], x)

    full = lambda shape: pl.BlockSpec(shape, lambda *_: (0,) * len(shape))
    table = pl.pallas_call(
        _table_body,
        out_shape=jax.ShapeDtypeStruct((TBL, HID), jnp.float32),
        in_specs=[full((1, HID)), full((1, HID)), full((HID, HID)),
                  full((1, HID)), full((HID, HID)), full((1, HID)),
                  full((1, 1))],
        out_specs=full((TBL, HID)),
    )(xcn_w1, xcn_b1.reshape(1, HID), xcn_w2, xcn_b2.reshape(1, HID),
      xcn_w3, xcn_b3.reshape(1, HID), beta.reshape(1, 1))

    nb = B // BLK
    out = pl.pallas_call(
        _main_body,
        grid=(nb,),
        out_shape=jax.ShapeDtypeStruct((B, 1), jnp.float32),
        in_specs=[
            pl.BlockSpec((BLK, 16), lambda i: (i, 0)),
            pl.BlockSpec((BLK, IN_CH), lambda i: (i, 0)),
            full((TBL, HID)),
            full((IN_CH, HID)), full((1, HID)),
            full((HID, HID)), full((1, HID)),
            full((HID, HID)), full((1, HID)),
            full((HID, 1)), full((1, 1)),
        ],
        out_specs=pl.BlockSpec((BLK, 1), lambda i: (i, 0)),
    )(counts, xij, table,
      xij_w1, xij_b1.reshape(1, HID), xij_w2, xij_b2.reshape(1, HID),
      lin_w1, lin_b1.reshape(1, HID), lin_w2, lin_b2.reshape(1, 1))
    return out


# speedup vs baseline: 6.6791x; 2.1622x over previous
import functools

import jax
import jax.numpy as jnp
from jax import lax
from jax.experimental import pallas as pl
from jax.experimental.pallas import tpu as pltpu
from jax.experimental.pallas import tpu_sc as plsc

N = 10000
B = 65536
IN_CH = 128
HID = 256
TBL = 128
BLK = 512

WRDS = 320
NWK = 32
PW = B // NWK
CH = 64
NCH = PW // CH


def _pair_body(bm_hbm, tar0_hbm, tar1_hbm, x_hbm, counts_hbm, xij_hbm,
               idx_i, idx_j, rows_i, rows_j, xi, xj, xij_buf, cnt_buf, sem):
    wid = lax.axis_index("s") * 2 + lax.axis_index("c")
    base = wid * PW

    def chunk(ch, carry):
        off = base + ch * CH
        pltpu.sync_copy(tar0_hbm.at[pl.ds(off, CH)], idx_i)
        pltpu.sync_copy(tar1_hbm.at[pl.ds(off, CH)], idx_j)
        h1 = pltpu.async_copy(bm_hbm.at[idx_i], rows_i, sem)
        h2 = pltpu.async_copy(bm_hbm.at[idx_j], rows_j, sem)
        h3 = pltpu.async_copy(x_hbm.at[idx_i], xi, sem)
        h4 = pltpu.async_copy(x_hbm.at[idx_j], xj, sem)
        h1.wait()
        h2.wait()
        h3.wait()
        h4.wait()

        def pair(p, c2):
            acc = jnp.zeros((16,), jnp.int32)
            for k in range(WRDS // 16):
                v = (rows_i[p, pl.ds(k * 16, 16)]
                     & rows_j[p, pl.ds(k * 16, 16)])
                v = v - (lax.shift_right_logical(v, 1) & 0x55555555)
                v = ((v & 0x33333333)
                     + (lax.shift_right_logical(v, 2) & 0x33333333))
                v = (v + lax.shift_right_logical(v, 4)) & 0x0F0F0F0F
                acc = acc + v
            cnt_buf[p, :] = lax.shift_right_logical(
                acc * jnp.int32(0x01010101), 24)
            for k in range(IN_CH // 16):
                s = pl.ds(k * 16, 16)
                xij_buf[p, s] = xi[p, s] * xj[p, s]
            return c2
        lax.fori_loop(0, CH, pair, 0)
        pltpu.sync_copy(xij_buf, xij_hbm.at[pl.ds(off, CH)])
        pltpu.sync_copy(cnt_buf, counts_hbm.at[pl.ds(off, CH)])
        return carry

    lax.fori_loop(0, NCH, chunk, 0)


def _sc_pairs(bm, tar0, tar1, x):
    mesh = plsc.VectorSubcoreMesh(core_axis_name="c", subcore_axis_name="s")
    f = pl.kernel(
        _pair_body,
        out_type=(jax.ShapeDtypeStruct((B, 16), jnp.int32),
                  jax.ShapeDtypeStruct((B, IN_CH), jnp.float32)),
        mesh=mesh,
        scratch_types=[
            pltpu.VMEM((CH,), jnp.int32),
            pltpu.VMEM((CH,), jnp.int32),
            pltpu.VMEM((CH, WRDS), jnp.int32),
            pltpu.VMEM((CH, WRDS), jnp.int32),
            pltpu.VMEM((CH, IN_CH), jnp.float32),
            pltpu.VMEM((CH, IN_CH), jnp.float32),
            pltpu.VMEM((CH, IN_CH), jnp.float32),
            pltpu.VMEM((CH, 16), jnp.int32),
            pltpu.SemaphoreType.DMA,
        ],
        compiler_params=pltpu.CompilerParams(use_tc_tiling_on_sc=False),
    )
    return f(bm, tar0, tar1, x)


def _table_body(w1, b1, w2, b2, w3, b3, beta, out_ref):
    c = lax.broadcasted_iota(jnp.int32, (TBL, 1), 0).astype(jnp.float32)
    h = jax.nn.relu(c * w1[...] + b1[...])
    h = jax.nn.relu(
        jax.lax.dot_general(h, w2[...], (((1,), (0,)), ((), ())),
                            preferred_element_type=jnp.float32) + b2[...])
    t = jax.lax.dot_general(h, w3[...], (((1,), (0,)), ((), ())),
                            preferred_element_type=jnp.float32) + b3[...]
    out_ref[...] = t * beta[...]


def _main_body(cnt_ref, xij_ref, table_ref, w1, b1, w2, b2, l1, lb1, l2, lb2,
               out_ref):
    xij = xij_ref[...]
    h = jax.nn.relu(
        jax.lax.dot_general(xij, w1[...], (((1,), (0,)), ((), ())),
                            preferred_element_type=jnp.float32) + b1[...])
    hij = jax.lax.dot_general(h, w2[...], (((1,), (0,)), ((), ())),
                              preferred_element_type=jnp.float32) + b2[...]
    c = jnp.sum(cnt_ref[...], axis=1, keepdims=True)
    onehot = (c == lax.broadcasted_iota(jnp.int32, (BLK, TBL), 1)
              ).astype(jnp.float32)
    hcn = jax.lax.dot_general(onehot, table_ref[...], (((1,), (0,)), ((), ())),
                              preferred_element_type=jnp.float32)
    z = hcn + hij
    h2 = jax.nn.relu(
        jax.lax.dot_general(z, l1[...], (((1,), (0,)), ((), ())),
                            preferred_element_type=jnp.float32) + lb1[...])
    out_ref[...] = jax.lax.dot_general(
        h2, l2[...], (((1,), (0,)), ((), ())),
        preferred_element_type=jnp.float32) + lb2[...]


EPAD = 320512
CE = 2048
ROWS_P = 1250
PLANE = ROWS_P * WRDS
DUMP = 4 * PLANE
CW = 2096
NCW = 12


def _build_body(srcp, dstp, out_hbm, spm, sbuf, dbuf, widx, wval, zbuf,
                pbuf, obuf):
    cid = lax.axis_index("c")
    sid = lax.axis_index("s")

    def z16(i, c):
        zbuf[pl.ds(i * 16, 16)] = jnp.zeros((16,), jnp.int32)
        return c
    lax.fori_loop(0, 4000 // 16, z16, 0)

    def do_pass(p, carry):
        rlo = cid * 5000 + p * ROWS_P

        def zc(i, c):
            pltpu.sync_copy(zbuf,
                            spm.at[pl.ds(sid * 100000 + i * 4000, 4000)])
            return c
        lax.fori_loop(0, 25, zc, 0)
        plsc.subcore_barrier()

        elim = sid * 20000 + 20000

        def chunk(ch, c):
            eoff = sid * 20000 + ch * CE
            pltpu.sync_copy(srcp.at[pl.ds(eoff, CE)], sbuf)
            pltpu.sync_copy(dstp.at[pl.ds(eoff, CE)], dbuf)

            def vec(v, c2):
                sl = pl.ds(v * 16, 16)
                s = sbuf[sl]
                d = dbuf[sl]
                pos = eoff + v * 16 + lax.iota(jnp.int32, 16)
                r = s - rlo
                inr = (r >= 0) & (r < ROWS_P) & (pos < elim)
                idx = (((d & 3) * ROWS_P + r) * WRDS
                       + lax.shift_right_logical(d, 5))
                idx = jnp.where(inr, idx, DUMP)
                val = jnp.where(
                    inr,
                    jnp.int32(1) << ((lax.shift_right_logical(d, 2) & 7) * 4),
                    0)
                j = v >> 3
                csl = pl.ds((v & 7) * 16, 16)
                widx[j, csl] = idx
                wval[j, csl] = val
                return c2
            lax.fori_loop(0, CE // 16, vec, 0)
            for j in range(16):
                pltpu.sync_copy(wval.at[j], spm.at[widx.at[j]], add=True)
            return c
        lax.fori_loop(0, 10, chunk, 0)
        plsc.subcore_barrier()

        def cchunk(k, c):
            poff = sid * 25000 + k * CW
            for q in range(4):
                pltpu.sync_copy(spm.at[pl.ds(q * PLANE + poff, CW)],
                                pbuf.at[q])

            def cvec(v, c2):
                sl = pl.ds(v * 16, 16)
                o = jnp.zeros((16,), jnp.int32)
                for q in range(4):
                    w = pbuf[q, sl]
                    nz = ((w | lax.shift_right_logical(w, 1)
                           | lax.shift_right_logical(w, 2)
                           | lax.shift_right_logical(w, 3)) & 0x11111111)
                    o = o | (nz << q)
                obuf[sl] = o
                return c2
            lax.fori_loop(0, CW // 16, cvec, 0)
            pltpu.sync_copy(
                obuf, out_hbm.at[cid, pl.ds(p * PLANE + poff, CW)])
            return c
        lax.fori_loop(0, NCW, cchunk, 0)
        plsc.subcore_barrier()
        return carry

    lax.fori_loop(0, 4, do_pass, 0)


def _sc_build(edge_index):
    srcp = jnp.concatenate(
        [edge_index[0], jnp.zeros((EPAD - 320000,), jnp.int32)])
    dstp = jnp.concatenate(
        [edge_index[1], jnp.zeros((EPAD - 320000,), jnp.int32)])
    mesh = plsc.VectorSubcoreMesh(core_axis_name="c", subcore_axis_name="s")
    f = pl.kernel(
        _build_body,
        out_type=jax.ShapeDtypeStruct((2, 1600160), jnp.int32),
        mesh=mesh,
        scratch_types=[
            pltpu.VMEM_SHARED((4 * PLANE + 160,), jnp.int32),
            pltpu.VMEM((CE,), jnp.int32),
            pltpu.VMEM((CE,), jnp.int32),
            pltpu.VMEM((16, 128), jnp.int32),
            pltpu.VMEM((16, 128), jnp.int32),
            pltpu.VMEM((4000,), jnp.int32),
            pltpu.VMEM((4, CW), jnp.int32),
            pltpu.VMEM((CW,), jnp.int32),
        ],
        compiler_params=pltpu.CompilerParams(use_tc_tiling_on_sc=False),
    )
    out = f(srcp, dstp)
    return jnp.concatenate(
        [out[0, :1600000], out[1, :1600000]]).reshape(N, WRDS)


def kernel(x, edge_index, tar_ei, beta, xcn_w1, xcn_b1, xcn_w2, xcn_b2,
           xcn_w3, xcn_b3, xij_w1, xij_b1, xij_w2, xij_b2,
           lin_w1, lin_b1, lin_w2, lin_b2):
    bm = _sc_build(edge_index)
    counts, xij = _sc_pairs(bm, tar_ei[0], tar_ei[1], x)

    full = lambda shape: pl.BlockSpec(shape, lambda *_: (0,) * len(shape))
    table = pl.pallas_call(
        _table_body,
        out_shape=jax.ShapeDtypeStruct((TBL, HID), jnp.float32),
        in_specs=[full((1, HID)), full((1, HID)), full((HID, HID)),
                  full((1, HID)), full((HID, HID)), full((1, HID)),
                  full((1, 1))],
        out_specs=full((TBL, HID)),
    )(xcn_w1, xcn_b1.reshape(1, HID), xcn_w2, xcn_b2.reshape(1, HID),
      xcn_w3, xcn_b3.reshape(1, HID), beta.reshape(1, 1))

    nb = B // BLK
    out = pl.pallas_call(
        _main_body,
        grid=(nb,),
        out_shape=jax.ShapeDtypeStruct((B, 1), jnp.float32),
        in_specs=[
            pl.BlockSpec((BLK, 16), lambda i: (i, 0)),
            pl.BlockSpec((BLK, IN_CH), lambda i: (i, 0)),
            full((TBL, HID)),
            full((IN_CH, HID)), full((1, HID)),
            full((HID, HID)), full((1, HID)),
            full((HID, HID)), full((1, HID)),
            full((HID, 1)), full((1, 1)),
        ],
        out_specs=pl.BlockSpec((BLK, 1), lambda i: (i, 0)),
    )(counts, xij, table,
      xij_w1, xij_b1.reshape(1, HID), xij_w2, xij_b2.reshape(1, HID),
      lin_w1, lin_b1.reshape(1, HID), lin_w2, lin_b2.reshape(1, 1))
    return out


# speedup vs baseline: 6.6796x; 1.0001x over previous
import functools

import jax
import jax.numpy as jnp
from jax import lax
from jax.experimental import pallas as pl
from jax.experimental.pallas import tpu as pltpu
from jax.experimental.pallas import tpu_sc as plsc

N = 10000
B = 65536
IN_CH = 128
HID = 256
TBL = 128
BLK = 512

WRDS = 320
NWK = 32
PW = B // NWK
CH = 64
NCH = PW // CH


def _pair_body(bm_hbm, tar0_hbm, tar1_hbm, x_hbm, counts_hbm, xij_hbm,
               idx_i, idx_j, idx_ib, idx_jb, rows_i, rows_j, xi, xj,
               xij_buf, cnt_buf, sem):
    wid = lax.axis_index("s") * 2 + lax.axis_index("c")
    base = wid * PW

    def chunk(ch, carry):
        off = base + ch * CH
        pltpu.sync_copy(tar0_hbm.at[pl.ds(off, CH)], idx_i)
        pltpu.sync_copy(tar1_hbm.at[pl.ds(off, CH)], idx_j)

        h1 = pltpu.async_copy(bm_hbm.at[idx_i], rows_i, sem)
        h2 = pltpu.async_copy(bm_hbm.at[idx_j], rows_j, sem)
        h3 = pltpu.async_copy(x_hbm.at[idx_i], xi, sem)
        h4 = pltpu.async_copy(x_hbm.at[idx_j], xj, sem)
        h1.wait()
        h2.wait()
        h3.wait()
        h4.wait()

        def pair(p, c2):
            acc = jnp.zeros((16,), jnp.int32)
            for k in range(WRDS // 16):
                v = (rows_i[p, pl.ds(k * 16, 16)]
                     & rows_j[p, pl.ds(k * 16, 16)])
                v = v - (lax.shift_right_logical(v, 1) & 0x55555555)
                v = ((v & 0x33333333)
                     + (lax.shift_right_logical(v, 2) & 0x33333333))
                v = (v + lax.shift_right_logical(v, 4)) & 0x0F0F0F0F
                acc = acc + v
            cnt_buf[p, :] = lax.shift_right_logical(
                acc * jnp.int32(0x01010101), 24)
            for k in range(IN_CH // 16):
                s = pl.ds(k * 16, 16)
                xij_buf[p, s] = xi[p, s] * xj[p, s]
            return c2
        lax.fori_loop(0, CH, pair, 0)
        pltpu.sync_copy(xij_buf, xij_hbm.at[pl.ds(off, CH)])
        pltpu.sync_copy(cnt_buf, counts_hbm.at[pl.ds(off, CH)])
        return carry

    lax.fori_loop(0, NCH, chunk, 0)


def _sc_pairs(bm, tar0, tar1, x):
    mesh = plsc.VectorSubcoreMesh(core_axis_name="c", subcore_axis_name="s")
    f = pl.kernel(
        _pair_body,
        out_type=(jax.ShapeDtypeStruct((B, 16), jnp.int32),
                  jax.ShapeDtypeStruct((B, IN_CH), jnp.float32)),
        mesh=mesh,
        scratch_types=[
            pltpu.VMEM((CH,), jnp.int32),
            pltpu.VMEM((CH,), jnp.int32),
            pltpu.VMEM((CH,), jnp.int32),
            pltpu.VMEM((CH,), jnp.int32),
            pltpu.VMEM((CH, WRDS), jnp.int32),
            pltpu.VMEM((CH, WRDS), jnp.int32),
            pltpu.VMEM((CH, IN_CH), jnp.float32),
            pltpu.VMEM((CH, IN_CH), jnp.float32),
            pltpu.VMEM((CH, IN_CH), jnp.float32),
            pltpu.VMEM((CH, 16), jnp.int32),
            pltpu.SemaphoreType.DMA,
        ],
        compiler_params=pltpu.CompilerParams(use_tc_tiling_on_sc=False),
    )
    return f(bm, tar0, tar1, x)


def _table_body(w1, b1, w2, b2, w3, b3, beta, out_ref):
    c = lax.broadcasted_iota(jnp.int32, (TBL, 1), 0).astype(jnp.float32)
    h = jax.nn.relu(c * w1[...] + b1[...])
    h = jax.nn.relu(
        jax.lax.dot_general(h, w2[...], (((1,), (0,)), ((), ())),
                            preferred_element_type=jnp.float32) + b2[...])
    t = jax.lax.dot_general(h, w3[...], (((1,), (0,)), ((), ())),
                            preferred_element_type=jnp.float32) + b3[...]
    out_ref[...] = t * beta[...]


def _main_body(cnt_ref, xij_ref, table_ref, w1, b1, w2, b2, l1, lb1, l2, lb2,
               out_ref):
    xij = xij_ref[...]
    h = jax.nn.relu(
        jax.lax.dot_general(xij, w1[...], (((1,), (0,)), ((), ())),
                            preferred_element_type=jnp.float32) + b1[...])
    hij = jax.lax.dot_general(h, w2[...], (((1,), (0,)), ((), ())),
                              preferred_element_type=jnp.float32) + b2[...]
    c = jnp.sum(cnt_ref[...], axis=1, keepdims=True)
    onehot = (c == lax.broadcasted_iota(jnp.int32, (BLK, TBL), 1)
              ).astype(jnp.float32)
    hcn = jax.lax.dot_general(onehot, table_ref[...], (((1,), (0,)), ((), ())),
                              preferred_element_type=jnp.float32)
    z = hcn + hij
    h2 = jax.nn.relu(
        jax.lax.dot_general(z, l1[...], (((1,), (0,)), ((), ())),
                            preferred_element_type=jnp.float32) + lb1[...])
    out_ref[...] = jax.lax.dot_general(
        h2, l2[...], (((1,), (0,)), ((), ())),
        preferred_element_type=jnp.float32) + lb2[...]


EPAD = 320512
CE = 2048
ROWS_P = 1250
PLANE = ROWS_P * WRDS
DUMP = 4 * PLANE
CW = 2096
NCW = 12


def _build_body(srcp, dstp, out_hbm, spm, sbuf, dbuf, widx, wval, zbuf,
                pbuf, obuf, sem):
    cid = lax.axis_index("c")
    sid = lax.axis_index("s")

    def z16(i, c):
        zbuf[pl.ds(i * 16, 16)] = jnp.zeros((16,), jnp.int32)
        return c
    lax.fori_loop(0, 4000 // 16, z16, 0)

    def do_pass(p, carry):
        rlo = cid * 5000 + p * ROWS_P

        def zc(i, c):
            pltpu.sync_copy(zbuf,
                            spm.at[pl.ds(sid * 100000 + i * 4000, 4000)])
            return c
        lax.fori_loop(0, 25, zc, 0)
        plsc.subcore_barrier()

        elim = sid * 20000 + 20000

        def chunk(ch, c):
            eoff = sid * 20000 + ch * CE
            pltpu.sync_copy(srcp.at[pl.ds(eoff, CE)], sbuf)
            pltpu.sync_copy(dstp.at[pl.ds(eoff, CE)], dbuf)

            for j in range(16):
                def vec(v8, c2):
                    v = j * 8 + v8
                    sl = pl.ds(v * 16, 16)
                    s = sbuf[sl]
                    d = dbuf[sl]
                    pos = eoff + v * 16 + lax.iota(jnp.int32, 16)
                    r = s - rlo
                    inr = (r >= 0) & (r < ROWS_P) & (pos < elim)
                    idx = (((d & 3) * ROWS_P + r) * WRDS
                           + lax.shift_right_logical(d, 5))
                    idx = jnp.where(inr, idx, DUMP)
                    val = jnp.where(
                        inr,
                        jnp.int32(1)
                        << ((lax.shift_right_logical(d, 2) & 7) * 4),
                        0)
                    csl = pl.ds(v8 * 16, 16)
                    widx[j, csl] = idx
                    wval[j, csl] = val
                    return c2
                lax.fori_loop(0, 8, vec, 0)
                pltpu.sync_copy(wval.at[j], spm.at[widx.at[j]], add=True)
            return c
        lax.fori_loop(0, 10, chunk, 0)
        plsc.subcore_barrier()

        def cchunk(k, c):
            poff = sid * 25000 + k * CW
            for q in range(4):
                pltpu.sync_copy(spm.at[pl.ds(q * PLANE + poff, CW)],
                                pbuf.at[q])

            def cvec(v, c2):
                sl = pl.ds(v * 16, 16)
                o = jnp.zeros((16,), jnp.int32)
                for q in range(4):
                    w = pbuf[q, sl]
                    nz = ((w | lax.shift_right_logical(w, 1)
                           | lax.shift_right_logical(w, 2)
                           | lax.shift_right_logical(w, 3)) & 0x11111111)
                    o = o | (nz << q)
                obuf[sl] = o
                return c2
            lax.fori_loop(0, CW // 16, cvec, 0)
            pltpu.sync_copy(
                obuf, out_hbm.at[cid, pl.ds(p * PLANE + poff, CW)])
            return c
        lax.fori_loop(0, NCW, cchunk, 0)
        plsc.subcore_barrier()
        return carry

    lax.fori_loop(0, 4, do_pass, 0)


def _sc_build(edge_index):
    srcp = jnp.concatenate(
        [edge_index[0], jnp.zeros((EPAD - 320000,), jnp.int32)])
    dstp = jnp.concatenate(
        [edge_index[1], jnp.zeros((EPAD - 320000,), jnp.int32)])
    mesh = plsc.VectorSubcoreMesh(core_axis_name="c", subcore_axis_name="s")
    f = pl.kernel(
        _build_body,
        out_type=jax.ShapeDtypeStruct((2, 1600320), jnp.int32),
        mesh=mesh,
        scratch_types=[
            pltpu.VMEM_SHARED((4 * PLANE + 160,), jnp.int32),
            pltpu.VMEM((CE,), jnp.int32),
            pltpu.VMEM((CE,), jnp.int32),
            pltpu.VMEM((16, 128), jnp.int32),
            pltpu.VMEM((16, 128), jnp.int32),
            pltpu.VMEM((4000,), jnp.int32),
            pltpu.VMEM((4, CW), jnp.int32),
            pltpu.VMEM((CW,), jnp.int32),
            pltpu.SemaphoreType.DMA,
        ],
        compiler_params=pltpu.CompilerParams(use_tc_tiling_on_sc=False),
    )
    out = f(srcp, dstp)
    return jnp.concatenate(
        [out[0, :1600000], out[1, :1600000]]).reshape(N, WRDS)


def kernel(x, edge_index, tar_ei, beta, xcn_w1, xcn_b1, xcn_w2, xcn_b2,
           xcn_w3, xcn_b3, xij_w1, xij_b1, xij_w2, xij_b2,
           lin_w1, lin_b1, lin_w2, lin_b2):
    bm = _sc_build(edge_index)
    counts, xij = _sc_pairs(bm, tar_ei[0], tar_ei[1], x)

    full = lambda shape: pl.BlockSpec(shape, lambda *_: (0,) * len(shape))
    table = pl.pallas_call(
        _table_body,
        out_shape=jax.ShapeDtypeStruct((TBL, HID), jnp.float32),
        in_specs=[full((1, HID)), full((1, HID)), full((HID, HID)),
                  full((1, HID)), full((HID, HID)), full((1, HID)),
                  full((1, 1))],
        out_specs=full((TBL, HID)),
    )(xcn_w1, xcn_b1.reshape(1, HID), xcn_w2, xcn_b2.reshape(1, HID),
      xcn_w3, xcn_b3.reshape(1, HID), beta.reshape(1, 1))

    nb = B // BLK
    out = pl.pallas_call(
        _main_body,
        grid=(nb,),
        out_shape=jax.ShapeDtypeStruct((B, 1), jnp.float32),
        in_specs=[
            pl.BlockSpec((BLK, 16), lambda i: (i, 0)),
            pl.BlockSpec((BLK, IN_CH), lambda i: (i, 0)),
            full((TBL, HID)),
            full((IN_CH, HID)), full((1, HID)),
            full((HID, HID)), full((1, HID)),
            full((HID, HID)), full((1, HID)),
            full((HID, 1)), full((1, 1)),
        ],
        out_specs=pl.BlockSpec((BLK, 1), lambda i: (i, 0)),
    )(counts, xij, table,
      xij_w1, xij_b1.reshape(1, HID), xij_w2, xij_b2.reshape(1, HID),
      lin_w1, lin_b1.reshape(1, HID), lin_w2, lin_b2.reshape(1, 1))
    return out


# speedup vs baseline: 6.7703x; 1.0136x over previous
import functools

import jax
import jax.numpy as jnp
from jax import lax
from jax.experimental import pallas as pl
from jax.experimental.pallas import tpu as pltpu
from jax.experimental.pallas import tpu_sc as plsc

N = 10000
B = 65536
IN_CH = 128
HID = 256
TBL = 128
BLK = 512

WRDS = 320
NWK = 32
PW = B // NWK
CH = 64
NCH = PW // CH


def _pair_body(bm_hbm, tar0_hbm, tar1_hbm, x_hbm, counts_hbm, xij_hbm,
               idx_i, idx_j, idx_ib, idx_jb, rows_i, rows_j, xi, xj,
               xij_buf, cnt_buf, sem):
    wid = lax.axis_index("s") * 2 + lax.axis_index("c")
    base = wid * PW

    def chunk(ch, carry):
        off = base + ch * CH
        pltpu.sync_copy(tar0_hbm.at[pl.ds(off, CH)], idx_i)
        pltpu.sync_copy(tar1_hbm.at[pl.ds(off, CH)], idx_j)

        h1 = pltpu.async_copy(bm_hbm.at[idx_i], rows_i, sem)
        h2 = pltpu.async_copy(bm_hbm.at[idx_j], rows_j, sem)
        h3 = pltpu.async_copy(x_hbm.at[idx_i], xi, sem)
        h4 = pltpu.async_copy(x_hbm.at[idx_j], xj, sem)
        h1.wait()
        h2.wait()
        h3.wait()
        h4.wait()

        def pair(p, c2):
            acc = jnp.zeros((16,), jnp.int32)
            for k in range(WRDS // 16):
                v = (rows_i[p, pl.ds(k * 16, 16)]
                     & rows_j[p, pl.ds(k * 16, 16)])
                v = v - (lax.shift_right_logical(v, 1) & 0x55555555)
                v = ((v & 0x33333333)
                     + (lax.shift_right_logical(v, 2) & 0x33333333))
                v = (v + lax.shift_right_logical(v, 4)) & 0x0F0F0F0F
                acc = acc + v
            cnt_buf[p, :] = lax.shift_right_logical(
                acc * jnp.int32(0x01010101), 24)
            for k in range(IN_CH // 16):
                s = pl.ds(k * 16, 16)
                xij_buf[p, s] = xi[p, s] * xj[p, s]
            return c2
        lax.fori_loop(0, CH, pair, 0)
        pltpu.sync_copy(xij_buf, xij_hbm.at[pl.ds(off, CH)])
        pltpu.sync_copy(cnt_buf, counts_hbm.at[pl.ds(off, CH)])
        return carry

    lax.fori_loop(0, NCH, chunk, 0)


def _sc_pairs(bm, tar0, tar1, x):
    mesh = plsc.VectorSubcoreMesh(core_axis_name="c", subcore_axis_name="s")
    f = pl.kernel(
        _pair_body,
        out_type=(jax.ShapeDtypeStruct((B, 16), jnp.int32),
                  jax.ShapeDtypeStruct((B, IN_CH), jnp.float32)),
        mesh=mesh,
        scratch_types=[
            pltpu.VMEM((CH,), jnp.int32),
            pltpu.VMEM((CH,), jnp.int32),
            pltpu.VMEM((CH,), jnp.int32),
            pltpu.VMEM((CH,), jnp.int32),
            pltpu.VMEM((CH, WRDS), jnp.int32),
            pltpu.VMEM((CH, WRDS), jnp.int32),
            pltpu.VMEM((CH, IN_CH), jnp.float32),
            pltpu.VMEM((CH, IN_CH), jnp.float32),
            pltpu.VMEM((CH, IN_CH), jnp.float32),
            pltpu.VMEM((CH, 16), jnp.int32),
            pltpu.SemaphoreType.DMA,
        ],
        compiler_params=pltpu.CompilerParams(use_tc_tiling_on_sc=False),
    )
    return f(bm, tar0, tar1, x)


def _table_body(w1, b1, w2, b2, w3, b3, beta, out_ref):
    c = lax.broadcasted_iota(jnp.int32, (TBL, 1), 0).astype(jnp.float32)
    h = jax.nn.relu(c * w1[...] + b1[...])
    h = jax.nn.relu(
        jax.lax.dot_general(h, w2[...], (((1,), (0,)), ((), ())),
                            preferred_element_type=jnp.float32) + b2[...])
    t = jax.lax.dot_general(h, w3[...], (((1,), (0,)), ((), ())),
                            preferred_element_type=jnp.float32) + b3[...]
    out_ref[...] = t * beta[...]


def _main_body(cnt_ref, xij_ref, table_ref, w1, b1, w2, b2, l1, lb1, l2, lb2,
               out_ref):
    xij = xij_ref[...]
    h = jax.nn.relu(
        jax.lax.dot_general(xij, w1[...], (((1,), (0,)), ((), ())),
                            preferred_element_type=jnp.float32) + b1[...])
    hij = jax.lax.dot_general(h, w2[...], (((1,), (0,)), ((), ())),
                              preferred_element_type=jnp.float32) + b2[...]
    c = jnp.sum(cnt_ref[...], axis=1, keepdims=True)
    onehot = (c == lax.broadcasted_iota(jnp.int32, (BLK, TBL), 1)
              ).astype(jnp.float32)
    hcn = jax.lax.dot_general(onehot, table_ref[...], (((1,), (0,)), ((), ())),
                              preferred_element_type=jnp.float32)
    z = hcn + hij
    h2 = jax.nn.relu(
        jax.lax.dot_general(z, l1[...], (((1,), (0,)), ((), ())),
                            preferred_element_type=jnp.float32) + lb1[...])
    out_ref[...] = jax.lax.dot_general(
        h2, l2[...], (((1,), (0,)), ((), ())),
        preferred_element_type=jnp.float32) + lb2[...]


EPAD = 320512
CE = 2048
ROWS_P = 1250
PLANE = ROWS_P * WRDS
DUMP = 4 * PLANE
CW = 2096
NCW = 12


def _build_body(srcp, dstp, out_hbm, spm, sbuf, dbuf, widx, wval, zbuf,
                pbuf, obuf, sem):
    cid = lax.axis_index("c")
    sid = lax.axis_index("s")

    def z16(i, c):
        zbuf[pl.ds(i * 16, 16)] = jnp.zeros((16,), jnp.int32)
        return c
    lax.fori_loop(0, 4000 // 16, z16, 0)

    def do_pass(p, carry):
        rlo = cid * 5000 + p * ROWS_P

        for g in range(5):
            hz = [pltpu.async_copy(
                zbuf,
                spm.at[pl.ds(sid * 100000 + (g * 5 + i) * 4000, 4000)], sem)
                for i in range(5)]
            for h in hz:
                h.wait()
        plsc.subcore_barrier()

        elim = sid * 20000 + 20000

        def chunk(ch, c):
            eoff = sid * 20000 + ch * CE
            pltpu.sync_copy(srcp.at[pl.ds(eoff, CE)], sbuf)
            pltpu.sync_copy(dstp.at[pl.ds(eoff, CE)], dbuf)

            hs = []
            for j in range(16):
                def vec(v8, c2):
                    v = j * 8 + v8
                    sl = pl.ds(v * 16, 16)
                    s = sbuf[sl]
                    d = dbuf[sl]
                    pos = eoff + v * 16 + lax.iota(jnp.int32, 16)
                    r = s - rlo
                    inr = (r >= 0) & (r < ROWS_P) & (pos < elim)
                    idx = (((d & 3) * ROWS_P + r) * WRDS
                           + lax.shift_right_logical(d, 5))
                    idx = jnp.where(inr, idx, DUMP)
                    val = jnp.where(
                        inr,
                        jnp.int32(1)
                        << ((lax.shift_right_logical(d, 2) & 7) * 4),
                        0)
                    csl = pl.ds(v8 * 16, 16)
                    widx[j, csl] = idx
                    wval[j, csl] = val
                    return c2
                lax.fori_loop(0, 8, vec, 0)
                hs.append(pltpu.async_copy(
                    wval.at[j], spm.at[widx.at[j]], sem, add=True))
            for h in hs:
                h.wait()
            return c
        lax.fori_loop(0, 10, chunk, 0)
        plsc.subcore_barrier()

        def cchunk(k, c):
            poff = sid * 25000 + k * CW
            hp = [pltpu.async_copy(
                spm.at[pl.ds(q * PLANE + poff, CW)], pbuf.at[q], sem)
                for q in range(4)]
            for h in hp:
                h.wait()

            def cvec(v, c2):
                sl = pl.ds(v * 16, 16)
                o = jnp.zeros((16,), jnp.int32)
                for q in range(4):
                    w = pbuf[q, sl]
                    nz = ((w | lax.shift_right_logical(w, 1)
                           | lax.shift_right_logical(w, 2)
                           | lax.shift_right_logical(w, 3)) & 0x11111111)
                    o = o | (nz << q)
                obuf[sl] = o
                return c2
            lax.fori_loop(0, CW // 16, cvec, 0)
            pltpu.sync_copy(
                obuf, out_hbm.at[cid, pl.ds(p * PLANE + poff, CW)])
            return c
        lax.fori_loop(0, NCW, cchunk, 0)
        plsc.subcore_barrier()
        return carry

    lax.fori_loop(0, 4, do_pass, 0)


def _sc_build(edge_index):
    srcp = jnp.concatenate(
        [edge_index[0], jnp.zeros((EPAD - 320000,), jnp.int32)])
    dstp = jnp.concatenate(
        [edge_index[1], jnp.zeros((EPAD - 320000,), jnp.int32)])
    mesh = plsc.VectorSubcoreMesh(core_axis_name="c", subcore_axis_name="s")
    f = pl.kernel(
        _build_body,
        out_type=jax.ShapeDtypeStruct((2, 1600320), jnp.int32),
        mesh=mesh,
        scratch_types=[
            pltpu.VMEM_SHARED((4 * PLANE + 160,), jnp.int32),
            pltpu.VMEM((CE,), jnp.int32),
            pltpu.VMEM((CE,), jnp.int32),
            pltpu.VMEM((16, 128), jnp.int32),
            pltpu.VMEM((16, 128), jnp.int32),
            pltpu.VMEM((4000,), jnp.int32),
            pltpu.VMEM((4, CW), jnp.int32),
            pltpu.VMEM((CW,), jnp.int32),
            pltpu.SemaphoreType.DMA,
        ],
        compiler_params=pltpu.CompilerParams(use_tc_tiling_on_sc=False),
    )
    out = f(srcp, dstp)
    return jnp.concatenate(
        [out[0, :1600000], out[1, :1600000]]).reshape(N, WRDS)


def kernel(x, edge_index, tar_ei, beta, xcn_w1, xcn_b1, xcn_w2, xcn_b2,
           xcn_w3, xcn_b3, xij_w1, xij_b1, xij_w2, xij_b2,
           lin_w1, lin_b1, lin_w2, lin_b2):
    bm = _sc_build(edge_index)
    counts, xij = _sc_pairs(bm, tar_ei[0], tar_ei[1], x)

    full = lambda shape: pl.BlockSpec(shape, lambda *_: (0,) * len(shape))
    table = pl.pallas_call(
        _table_body,
        out_shape=jax.ShapeDtypeStruct((TBL, HID), jnp.float32),
        in_specs=[full((1, HID)), full((1, HID)), full((HID, HID)),
                  full((1, HID)), full((HID, HID)), full((1, HID)),
                  full((1, 1))],
        out_specs=full((TBL, HID)),
    )(xcn_w1, xcn_b1.reshape(1, HID), xcn_w2, xcn_b2.reshape(1, HID),
      xcn_w3, xcn_b3.reshape(1, HID), beta.reshape(1, 1))

    nb = B // BLK
    out = pl.pallas_call(
        _main_body,
        grid=(nb,),
        out_shape=jax.ShapeDtypeStruct((B, 1), jnp.float32),
        in_specs=[
            pl.BlockSpec((BLK, 16), lambda i: (i, 0)),
            pl.BlockSpec((BLK, IN_CH), lambda i: (i, 0)),
            full((TBL, HID)),
            full((IN_CH, HID)), full((1, HID)),
            full((HID, HID)), full((1, HID)),
            full((HID, HID)), full((1, HID)),
            full((HID, 1)), full((1, 1)),
        ],
        out_specs=pl.BlockSpec((BLK, 1), lambda i: (i, 0)),
    )(counts, xij, table,
      xij_w1, xij_b1.reshape(1, HID), xij_w2, xij_b2.reshape(1, HID),
      lin_w1, lin_b1.reshape(1, HID), lin_w2, lin_b2.reshape(1, 1))
    return out


# speedup vs baseline: 11.7020x; 1.7284x over previous
import functools

import jax
import jax.numpy as jnp
from jax import lax
from jax.experimental import pallas as pl
from jax.experimental.pallas import tpu as pltpu
from jax.experimental.pallas import tpu_sc as plsc

N = 10000
B = 65536
IN_CH = 128
HID = 256
TBL = 128
BLK = 512

WRDS = 320
NWK = 32
PW = B // NWK
CH = 64
NCH = PW // CH


def _pair_body(bm_hbm, tar0_hbm, tar1_hbm, x_hbm, counts_hbm, xij_hbm,
               idx_i, idx_j, idx_ib, idx_jb, rows_i, rows_j, xi, xj,
               xij_buf, cnt_buf, sem):
    wid = lax.axis_index("s") * 2 + lax.axis_index("c")
    base = wid * PW

    def chunk(ch, carry):
        off = base + ch * CH
        pltpu.sync_copy(tar0_hbm.at[pl.ds(off, CH)], idx_i)
        pltpu.sync_copy(tar1_hbm.at[pl.ds(off, CH)], idx_j)

        h1 = pltpu.async_copy(bm_hbm.at[idx_i], rows_i, sem)
        h2 = pltpu.async_copy(bm_hbm.at[idx_j], rows_j, sem)
        h3 = pltpu.async_copy(x_hbm.at[idx_i], xi, sem)
        h4 = pltpu.async_copy(x_hbm.at[idx_j], xj, sem)
        h1.wait()
        h2.wait()
        h3.wait()
        h4.wait()

        def pair(p, c2):
            acc = jnp.zeros((16,), jnp.int32)
            for k in range(WRDS // 16):
                v = (rows_i[p, pl.ds(k * 16, 16)]
                     & rows_j[p, pl.ds(k * 16, 16)])
                v = v - (lax.shift_right_logical(v, 1) & 0x55555555)
                v = ((v & 0x33333333)
                     + (lax.shift_right_logical(v, 2) & 0x33333333))
                v = (v + lax.shift_right_logical(v, 4)) & 0x0F0F0F0F
                acc = acc + v
            cnt_buf[p, :] = lax.shift_right_logical(
                acc * jnp.int32(0x01010101), 24)
            for k in range(IN_CH // 16):
                s = pl.ds(k * 16, 16)
                xij_buf[p, s] = xi[p, s] * xj[p, s]
            return c2
        lax.fori_loop(0, CH, pair, 0)
        pltpu.sync_copy(xij_buf, xij_hbm.at[pl.ds(off, CH)])
        pltpu.sync_copy(cnt_buf, counts_hbm.at[pl.ds(off, CH)])
        return carry

    lax.fori_loop(0, NCH, chunk, 0)


def _sc_pairs(bm, tar0, tar1, x):
    mesh = plsc.VectorSubcoreMesh(core_axis_name="c", subcore_axis_name="s")
    f = pl.kernel(
        _pair_body,
        out_type=(jax.ShapeDtypeStruct((B, 16), jnp.int32),
                  jax.ShapeDtypeStruct((B, IN_CH), jnp.float32)),
        mesh=mesh,
        scratch_types=[
            pltpu.VMEM((CH,), jnp.int32),
            pltpu.VMEM((CH,), jnp.int32),
            pltpu.VMEM((CH,), jnp.int32),
            pltpu.VMEM((CH,), jnp.int32),
            pltpu.VMEM((CH, WRDS), jnp.int32),
            pltpu.VMEM((CH, WRDS), jnp.int32),
            pltpu.VMEM((CH, IN_CH), jnp.float32),
            pltpu.VMEM((CH, IN_CH), jnp.float32),
            pltpu.VMEM((CH, IN_CH), jnp.float32),
            pltpu.VMEM((CH, 16), jnp.int32),
            pltpu.SemaphoreType.DMA,
        ],
        compiler_params=pltpu.CompilerParams(use_tc_tiling_on_sc=False),
    )
    return f(bm, tar0, tar1, x)


def _table_body(w1, b1, w2, b2, w3, b3, beta, out_ref):
    c = lax.broadcasted_iota(jnp.int32, (TBL, 1), 0).astype(jnp.float32)
    h = jax.nn.relu(c * w1[...] + b1[...])
    h = jax.nn.relu(
        jax.lax.dot_general(h, w2[...], (((1,), (0,)), ((), ())),
                            preferred_element_type=jnp.float32) + b2[...])
    t = jax.lax.dot_general(h, w3[...], (((1,), (0,)), ((), ())),
                            preferred_element_type=jnp.float32) + b3[...]
    out_ref[...] = t * beta[...]


def _main_body(cnt_ref, xij_ref, table_ref, w1, b1, w2, b2, l1, lb1, l2, lb2,
               out_ref):
    xij = xij_ref[...]
    h = jax.nn.relu(
        jax.lax.dot_general(xij, w1[...], (((1,), (0,)), ((), ())),
                            preferred_element_type=jnp.float32) + b1[...])
    hij = jax.lax.dot_general(h, w2[...], (((1,), (0,)), ((), ())),
                              preferred_element_type=jnp.float32) + b2[...]
    c = jnp.sum(cnt_ref[...], axis=1, keepdims=True)
    onehot = (c == lax.broadcasted_iota(jnp.int32, (BLK, TBL), 1)
              ).astype(jnp.float32)
    hcn = jax.lax.dot_general(onehot, table_ref[...], (((1,), (0,)), ((), ())),
                              preferred_element_type=jnp.float32)
    z = hcn + hij
    h2 = jax.nn.relu(
        jax.lax.dot_general(z, l1[...], (((1,), (0,)), ((), ())),
                            preferred_element_type=jnp.float32) + lb1[...])
    out_ref[...] = jax.lax.dot_general(
        h2, l2[...], (((1,), (0,)), ((), ())),
        preferred_element_type=jnp.float32) + lb2[...]


EPAD = 320512
CE = 2048
ROWS_P = 1250
PLANE = ROWS_P * WRDS
DUMP = 4 * PLANE
CW = 2096
NCW = 12


def _build_body(srcp, dstp, out_hbm, spm, sbuf, dbuf, widx, wval, zbuf,
                pbuf, obuf, sem):
    cid = lax.axis_index("c")
    sid = lax.axis_index("s")

    def z16(i, c):
        zbuf[pl.ds(i * 16, 16)] = jnp.zeros((16,), jnp.int32)
        return c
    lax.fori_loop(0, 4000 // 16, z16, 0)

    def do_pass(p, carry):
        rlo = cid * 5000 + p * ROWS_P

        for g in range(5):
            hz = [pltpu.async_copy(
                zbuf,
                spm.at[pl.ds(sid * 100000 + (g * 5 + i) * 4000, 4000)], sem)
                for i in range(5)]
            for h in hz:
                h.wait()
        plsc.subcore_barrier()

        elim = sid * 20000 + 20000

        def chunk(ch, c):
            eoff = sid * 20000 + ch * CE
            pltpu.sync_copy(srcp.at[pl.ds(eoff, CE)], sbuf)
            pltpu.sync_copy(dstp.at[pl.ds(eoff, CE)], dbuf)

            hs = []
            for j in range(16):
                def vec(v8, c2):
                    v = j * 8 + v8
                    sl = pl.ds(v * 16, 16)
                    s = sbuf[sl]
                    d = dbuf[sl]
                    pos = eoff + v * 16 + lax.iota(jnp.int32, 16)
                    r = s - rlo
                    inr = (r >= 0) & (r < ROWS_P) & (pos < elim)
                    idx = (((d & 3) * ROWS_P + r) * WRDS
                           + lax.shift_right_logical(d, 5))
                    idx = jnp.where(inr, idx, DUMP + (pos & 127))
                    val = jnp.where(
                        inr,
                        jnp.int32(1)
                        << ((lax.shift_right_logical(d, 2) & 7) * 4),
                        0)
                    csl = pl.ds(v8 * 16, 16)
                    widx[j, csl] = idx
                    wval[j, csl] = val
                    return c2
                lax.fori_loop(0, 8, vec, 0)
                hs.append(pltpu.async_copy(
                    wval.at[j], spm.at[widx.at[j]], sem, add=True))
            for h in hs:
                h.wait()
            return c
        lax.fori_loop(0, 10, chunk, 0)
        plsc.subcore_barrier()

        def cchunk(k, c):
            poff = sid * 25000 + k * CW
            hp = [pltpu.async_copy(
                spm.at[pl.ds(q * PLANE + poff, CW)], pbuf.at[q], sem)
                for q in range(4)]
            for h in hp:
                h.wait()

            def cvec(v, c2):
                sl = pl.ds(v * 16, 16)
                o = jnp.zeros((16,), jnp.int32)
                for q in range(4):
                    w = pbuf[q, sl]
                    nz = ((w | lax.shift_right_logical(w, 1)
                           | lax.shift_right_logical(w, 2)
                           | lax.shift_right_logical(w, 3)) & 0x11111111)
                    o = o | (nz << q)
                obuf[sl] = o
                return c2
            lax.fori_loop(0, CW // 16, cvec, 0)
            pltpu.sync_copy(
                obuf, out_hbm.at[cid, pl.ds(p * PLANE + poff, CW)])
            return c
        lax.fori_loop(0, NCW, cchunk, 0)
        plsc.subcore_barrier()
        return carry

    lax.fori_loop(0, 4, do_pass, 0)


def _sc_build(edge_index):
    srcp = jnp.concatenate(
        [edge_index[0], jnp.zeros((EPAD - 320000,), jnp.int32)])
    dstp = jnp.concatenate(
        [edge_index[1], jnp.zeros((EPAD - 320000,), jnp.int32)])
    mesh = plsc.VectorSubcoreMesh(core_axis_name="c", subcore_axis_name="s")
    f = pl.kernel(
        _build_body,
        out_type=jax.ShapeDtypeStruct((2, 1600320), jnp.int32),
        mesh=mesh,
        scratch_types=[
            pltpu.VMEM_SHARED((4 * PLANE + 160,), jnp.int32),
            pltpu.VMEM((CE,), jnp.int32),
            pltpu.VMEM((CE,), jnp.int32),
            pltpu.VMEM((16, 128), jnp.int32),
            pltpu.VMEM((16, 128), jnp.int32),
            pltpu.VMEM((4000,), jnp.int32),
            pltpu.VMEM((4, CW), jnp.int32),
            pltpu.VMEM((CW,), jnp.int32),
            pltpu.SemaphoreType.DMA,
        ],
        compiler_params=pltpu.CompilerParams(use_tc_tiling_on_sc=False),
    )
    out = f(srcp, dstp)
    return jnp.concatenate(
        [out[0, :1600000], out[1, :1600000]]).reshape(N, WRDS)


def kernel(x, edge_index, tar_ei, beta, xcn_w1, xcn_b1, xcn_w2, xcn_b2,
           xcn_w3, xcn_b3, xij_w1, xij_b1, xij_w2, xij_b2,
           lin_w1, lin_b1, lin_w2, lin_b2):
    bm = _sc_build(edge_index)
    counts, xij = _sc_pairs(bm, tar_ei[0], tar_ei[1], x)

    full = lambda shape: pl.BlockSpec(shape, lambda *_: (0,) * len(shape))
    table = pl.pallas_call(
        _table_body,
        out_shape=jax.ShapeDtypeStruct((TBL, HID), jnp.float32),
        in_specs=[full((1, HID)), full((1, HID)), full((HID, HID)),
                  full((1, HID)), full((HID, HID)), full((1, HID)),
                  full((1, 1))],
        out_specs=full((TBL, HID)),
    )(xcn_w1, xcn_b1.reshape(1, HID), xcn_w2, xcn_b2.reshape(1, HID),
      xcn_w3, xcn_b3.reshape(1, HID), beta.reshape(1, 1))

    nb = B // BLK
    out = pl.pallas_call(
        _main_body,
        grid=(nb,),
        out_shape=jax.ShapeDtypeStruct((B, 1), jnp.float32),
        in_specs=[
            pl.BlockSpec((BLK, 16), lambda i: (i, 0)),
            pl.BlockSpec((BLK, IN_CH), lambda i: (i, 0)),
            full((TBL, HID)),
            full((IN_CH, HID)), full((1, HID)),
            full((HID, HID)), full((1, HID)),
            full((HID, HID)), full((1, HID)),
            full((HID, 1)), full((1, 1)),
        ],
        out_specs=pl.BlockSpec((BLK, 1), lambda i: (i, 0)),
    )(counts, xij, table,
      xij_w1, xij_b1.reshape(1, HID), xij_w2, xij_b2.reshape(1, HID),
      lin_w1, lin_b1.reshape(1, HID), lin_w2, lin_b2.reshape(1, 1))
    return out


# speedup vs baseline: 13.2376x; 1.1312x over previous
import functools

import jax
import jax.numpy as jnp
from jax import lax
from jax.experimental import pallas as pl
from jax.experimental.pallas import tpu as pltpu
from jax.experimental.pallas import tpu_sc as plsc

N = 10000
B = 65536
IN_CH = 128
HID = 256
TBL = 128
BLK = 512

WRDS = 320
NWK = 32
PW = B // NWK
CH = 32
NCH = PW // CH


def _pair_body(bm_hbm, tar0_hbm, tar1_hbm, tar0b_hbm, tar1b_hbm, x_hbm,
               counts_hbm, xij_hbm,
               idx_i, idx_j, idx_ib, idx_jb, rows_i, rows_j, xi, xj,
               xij_buf, cnt_buf, sem):
    wid = lax.axis_index("s") * 2 + lax.axis_index("c")
    base = wid * PW

    def stage_and_fire(c, s):
        off = base + c * CH
        pltpu.sync_copy(tar0b_hbm.at[pl.ds(off, CH)], idx_ib.at[s])
        pltpu.sync_copy(tar1b_hbm.at[pl.ds(off, CH)], idx_jb.at[s])
        pltpu.sync_copy(tar0_hbm.at[pl.ds(off, CH)], idx_i.at[s])
        pltpu.sync_copy(tar1_hbm.at[pl.ds(off, CH)], idx_j.at[s])
        pltpu.async_copy(bm_hbm.at[idx_ib.at[s]], rows_i.at[s], sem)
        pltpu.async_copy(bm_hbm.at[idx_jb.at[s]], rows_j.at[s], sem)
        pltpu.async_copy(x_hbm.at[idx_i.at[s]], xi.at[s], sem)
        pltpu.async_copy(x_hbm.at[idx_j.at[s]], xj.at[s], sem)

    def drain(s):
        pltpu.make_async_copy(bm_hbm.at[pl.ds(0, CH)], rows_i.at[s],
                              sem).wait()
        pltpu.make_async_copy(bm_hbm.at[pl.ds(0, CH)], rows_j.at[s],
                              sem).wait()
        pltpu.make_async_copy(x_hbm.at[pl.ds(0, CH)], xi.at[s], sem).wait()
        pltpu.make_async_copy(x_hbm.at[pl.ds(0, CH)], xj.at[s], sem).wait()

    stage_and_fire(0, 0)

    def chunk(ch, carry):
        off = base + ch * CH
        cur = ch & 1
        nxt = (ch + 1) & 1
        c_next = jnp.where(ch < NCH - 1, ch + 1, 0)
        stage_and_fire(c_next, nxt)
        drain(cur)

        def pair(p, c2):
            acc = jnp.zeros((16,), jnp.int32)
            for k in range(WRDS // 16):
                v = (rows_i[cur, p, pl.ds(k * 16, 16)]
                     & rows_j[cur, p, pl.ds(k * 16, 16)])
                v = v - (lax.shift_right_logical(v, 1) & 0x55555555)
                v = ((v & 0x33333333)
                     + (lax.shift_right_logical(v, 2) & 0x33333333))
                v = (v + lax.shift_right_logical(v, 4)) & 0x0F0F0F0F
                acc = acc + v
            cnt_buf[p, :] = lax.shift_right_logical(
                acc * jnp.int32(0x01010101), 24)
            for k in range(IN_CH // 16):
                s = pl.ds(k * 16, 16)
                xij_buf[p, s] = xi[cur, p, s] * xj[cur, p, s]
            return c2
        lax.fori_loop(0, CH, pair, 0)
        pltpu.sync_copy(xij_buf, xij_hbm.at[pl.ds(off, CH)])
        pltpu.sync_copy(cnt_buf, counts_hbm.at[pl.ds(off, CH)])
        return carry

    lax.fori_loop(0, NCH, chunk, 0)
    drain(0)


def _sc_pairs(bm, tar0, tar1, tar0b, tar1b, x):
    mesh = plsc.VectorSubcoreMesh(core_axis_name="c", subcore_axis_name="s")
    f = pl.kernel(
        _pair_body,
        out_type=(jax.ShapeDtypeStruct((B, 16), jnp.int32),
                  jax.ShapeDtypeStruct((B, IN_CH), jnp.float32)),
        mesh=mesh,
        scratch_types=[
            pltpu.VMEM((2, CH), jnp.int32),
            pltpu.VMEM((2, CH), jnp.int32),
            pltpu.VMEM((2, CH), jnp.int32),
            pltpu.VMEM((2, CH), jnp.int32),
            pltpu.VMEM((2, CH, WRDS), jnp.int32),
            pltpu.VMEM((2, CH, WRDS), jnp.int32),
            pltpu.VMEM((2, CH, IN_CH), jnp.float32),
            pltpu.VMEM((2, CH, IN_CH), jnp.float32),
            pltpu.VMEM((CH, IN_CH), jnp.float32),
            pltpu.VMEM((CH, 16), jnp.int32),
            pltpu.SemaphoreType.DMA,
        ],
        compiler_params=pltpu.CompilerParams(use_tc_tiling_on_sc=False),
    )
    return f(bm, tar0, tar1, tar0b, tar1b, x)


def _table_body(w1, b1, w2, b2, w3, b3, beta, out_ref):
    c = lax.broadcasted_iota(jnp.int32, (TBL, 1), 0).astype(jnp.float32)
    h = jax.nn.relu(c * w1[...] + b1[...])
    h = jax.nn.relu(
        jax.lax.dot_general(h, w2[...], (((1,), (0,)), ((), ())),
                            preferred_element_type=jnp.float32) + b2[...])
    t = jax.lax.dot_general(h, w3[...], (((1,), (0,)), ((), ())),
                            preferred_element_type=jnp.float32) + b3[...]
    out_ref[...] = t * beta[...]


def _main_body(cnt_ref, xij_ref, table_ref, w1, b1, w2, b2, l1, lb1, l2, lb2,
               out_ref):
    xij = xij_ref[...]
    h = jax.nn.relu(
        jax.lax.dot_general(xij, w1[...], (((1,), (0,)), ((), ())),
                            preferred_element_type=jnp.float32) + b1[...])
    hij = jax.lax.dot_general(h, w2[...], (((1,), (0,)), ((), ())),
                              preferred_element_type=jnp.float32) + b2[...]
    c = jnp.sum(cnt_ref[...], axis=1, keepdims=True)
    onehot = (c == lax.broadcasted_iota(jnp.int32, (BLK, TBL), 1)
              ).astype(jnp.float32)
    hcn = jax.lax.dot_general(onehot, table_ref[...], (((1,), (0,)), ((), ())),
                              preferred_element_type=jnp.float32)
    z = hcn + hij
    h2 = jax.nn.relu(
        jax.lax.dot_general(z, l1[...], (((1,), (0,)), ((), ())),
                            preferred_element_type=jnp.float32) + lb1[...])
    out_ref[...] = jax.lax.dot_general(
        h2, l2[...], (((1,), (0,)), ((), ())),
        preferred_element_type=jnp.float32) + lb2[...]


EPAD = 320512
CE = 2048
ROWS_P = 1250
PLANE = ROWS_P * WRDS
DUMP = 4 * PLANE
CW = 2096
NCW = 12


def _build_body(srcp, dstp, out_hbm, spm, sbuf, dbuf, widx, wval, zbuf,
                pbuf, obuf, sem):
    cid = lax.axis_index("c")
    sid = lax.axis_index("s")

    def z16(i, c):
        zbuf[pl.ds(i * 16, 16)] = jnp.zeros((16,), jnp.int32)
        return c
    lax.fori_loop(0, 4000 // 16, z16, 0)

    def do_pass(p, carry):
        rlo = cid * 5000 + p * ROWS_P

        for g in range(5):
            hz = [pltpu.async_copy(
                zbuf,
                spm.at[pl.ds(sid * 100000 + (g * 5 + i) * 4000, 4000)], sem)
                for i in range(5)]
            for h in hz:
                h.wait()
        plsc.subcore_barrier()

        elim = sid * 20000 + 20000

        def chunk(ch, c):
            eoff = sid * 20000 + ch * CE
            pltpu.sync_copy(srcp.at[pl.ds(eoff, CE)], sbuf)
            pltpu.sync_copy(dstp.at[pl.ds(eoff, CE)], dbuf)

            hs = []
            for j in range(16):
                def vec(v8, c2):
                    v = j * 8 + v8
                    sl = pl.ds(v * 16, 16)
                    s = sbuf[sl]
                    d = dbuf[sl]
                    pos = eoff + v * 16 + lax.iota(jnp.int32, 16)
                    r = s - rlo
                    inr = (r >= 0) & (r < ROWS_P) & (pos < elim)
                    idx = (((d & 3) * ROWS_P + r) * WRDS
                           + lax.shift_right_logical(d, 5))
                    idx = jnp.where(inr, idx, DUMP + (pos & 127))
                    val = jnp.where(
                        inr,
                        jnp.int32(1)
                        << ((lax.shift_right_logical(d, 2) & 7) * 4),
                        0)
                    csl = pl.ds(v8 * 16, 16)
                    widx[j, csl] = idx
                    wval[j, csl] = val
                    return c2
                lax.fori_loop(0, 8, vec, 0)
                hs.append(pltpu.async_copy(
                    wval.at[j], spm.at[widx.at[j]], sem, add=True))
            for h in hs:
                h.wait()
            return c
        lax.fori_loop(0, 10, chunk, 0)
        plsc.subcore_barrier()

        def cchunk(k, c):
            poff = sid * 25000 + k * CW
            hp = [pltpu.async_copy(
                spm.at[pl.ds(q * PLANE + poff, CW)], pbuf.at[q], sem)
                for q in range(4)]
            for h in hp:
                h.wait()

            def cvec(v, c2):
                sl = pl.ds(v * 16, 16)
                o = jnp.zeros((16,), jnp.int32)
                for q in range(4):
                    w = pbuf[q, sl]
                    nz = ((w | lax.shift_right_logical(w, 1)
                           | lax.shift_right_logical(w, 2)
                           | lax.shift_right_logical(w, 3)) & 0x11111111)
                    o = o | (nz << q)
                obuf[sl] = o
                return c2
            lax.fori_loop(0, CW // 16, cvec, 0)
            pltpu.sync_copy(
                obuf, out_hbm.at[cid, pl.ds(p * PLANE + poff, CW)])
            return c
        lax.fori_loop(0, NCW, cchunk, 0)
        plsc.subcore_barrier()
        return carry

    lax.fori_loop(0, 4, do_pass, 0)


def _sc_build(edge_index):
    srcp = jnp.concatenate(
        [edge_index[0], jnp.zeros((EPAD - 320000,), jnp.int32)])
    dstp = jnp.concatenate(
        [edge_index[1], jnp.zeros((EPAD - 320000,), jnp.int32)])
    mesh = plsc.VectorSubcoreMesh(core_axis_name="c", subcore_axis_name="s")
    f = pl.kernel(
        _build_body,
        out_type=jax.ShapeDtypeStruct((2, 1600320), jnp.int32),
        mesh=mesh,
        scratch_types=[
            pltpu.VMEM_SHARED((4 * PLANE + 160,), jnp.int32),
            pltpu.VMEM((CE,), jnp.int32),
            pltpu.VMEM((CE,), jnp.int32),
            pltpu.VMEM((16, 128), jnp.int32),
            pltpu.VMEM((16, 128), jnp.int32),
            pltpu.VMEM((4000,), jnp.int32),
            pltpu.VMEM((4, CW), jnp.int32),
            pltpu.VMEM((CW,), jnp.int32),
            pltpu.SemaphoreType.DMA,
        ],
        compiler_params=pltpu.CompilerParams(use_tc_tiling_on_sc=False),
    )
    out = f(srcp, dstp)
    return out.reshape(10002, WRDS)


def kernel(x, edge_index, tar_ei, beta, xcn_w1, xcn_b1, xcn_w2, xcn_b2,
           xcn_w3, xcn_b3, xij_w1, xij_b1, xij_w2, xij_b2,
           lin_w1, lin_b1, lin_w2, lin_b2):
    bm = _sc_build(edge_index)
    tar0, tar1 = tar_ei[0], tar_ei[1]
    tar0b = tar0 + (tar0 >= 5000).astype(jnp.int32)
    tar1b = tar1 + (tar1 >= 5000).astype(jnp.int32)
    counts, xij = _sc_pairs(bm, tar0, tar1, tar0b, tar1b, x)

    full = lambda shape: pl.BlockSpec(shape, lambda *_: (0,) * len(shape))
    table = pl.pallas_call(
        _table_body,
        out_shape=jax.ShapeDtypeStruct((TBL, HID), jnp.float32),
        in_specs=[full((1, HID)), full((1, HID)), full((HID, HID)),
                  full((1, HID)), full((HID, HID)), full((1, HID)),
                  full((1, 1))],
        out_specs=full((TBL, HID)),
    )(xcn_w1, xcn_b1.reshape(1, HID), xcn_w2, xcn_b2.reshape(1, HID),
      xcn_w3, xcn_b3.reshape(1, HID), beta.reshape(1, 1))

    nb = B // BLK
    out = pl.pallas_call(
        _main_body,
        grid=(nb,),
        out_shape=jax.ShapeDtypeStruct((B, 1), jnp.float32),
        in_specs=[
            pl.BlockSpec((BLK, 16), lambda i: (i, 0)),
            pl.BlockSpec((BLK, IN_CH), lambda i: (i, 0)),
            full((TBL, HID)),
            full((IN_CH, HID)), full((1, HID)),
            full((HID, HID)), full((1, HID)),
            full((HID, HID)), full((1, HID)),
            full((HID, 1)), full((1, 1)),
        ],
        out_specs=pl.BlockSpec((BLK, 1), lambda i: (i, 0)),
    )(counts, xij, table,
      xij_w1, xij_b1.reshape(1, HID), xij_w2, xij_b2.reshape(1, HID),
      lin_w1, lin_b1.reshape(1, HID), lin_w2, lin_b2.reshape(1, 1))
    return out


# speedup vs baseline: 13.4724x; 1.0177x over previous
import functools

import jax
import jax.numpy as jnp
from jax import lax
from jax.experimental import pallas as pl
from jax.experimental.pallas import tpu as pltpu
from jax.experimental.pallas import tpu_sc as plsc

N = 10000
B = 65536
IN_CH = 128
HID = 256
TBL = 128
BLK = 512

WRDS = 320
NWK = 32
PW = B // NWK
CH = 32
NCH = PW // CH


def _pair_body(bm_hbm, tar0_hbm, tar1_hbm, tar0b_hbm, tar1b_hbm, x_hbm,
               counts_hbm, xij_hbm,
               idx_i, idx_j, idx_ib, idx_jb, rows_i, rows_j, xi, xj,
               xij_buf, cnt_buf, sem):
    wid = lax.axis_index("s") * 2 + lax.axis_index("c")
    base = wid * PW

    def stage_and_fire(c, s):
        off = base + c * CH
        pltpu.sync_copy(tar0b_hbm.at[pl.ds(off, CH)], idx_ib.at[s])
        pltpu.sync_copy(tar1b_hbm.at[pl.ds(off, CH)], idx_jb.at[s])
        pltpu.sync_copy(tar0_hbm.at[pl.ds(off, CH)], idx_i.at[s])
        pltpu.sync_copy(tar1_hbm.at[pl.ds(off, CH)], idx_j.at[s])
        pltpu.async_copy(bm_hbm.at[idx_ib.at[s]], rows_i.at[s], sem)
        pltpu.async_copy(bm_hbm.at[idx_jb.at[s]], rows_j.at[s], sem)
        pltpu.async_copy(x_hbm.at[idx_i.at[s]], xi.at[s], sem)
        pltpu.async_copy(x_hbm.at[idx_j.at[s]], xj.at[s], sem)

    def drain(s):
        pltpu.make_async_copy(bm_hbm.at[pl.ds(0, CH)], rows_i.at[s],
                              sem).wait()
        pltpu.make_async_copy(bm_hbm.at[pl.ds(0, CH)], rows_j.at[s],
                              sem).wait()
        pltpu.make_async_copy(x_hbm.at[pl.ds(0, CH)], xi.at[s], sem).wait()
        pltpu.make_async_copy(x_hbm.at[pl.ds(0, CH)], xj.at[s], sem).wait()

    stage_and_fire(0, 0)

    def chunk(ch, carry):
        off = base + ch * CH
        cur = ch & 1
        nxt = (ch + 1) & 1
        c_next = jnp.where(ch < NCH - 1, ch + 1, 0)
        stage_and_fire(c_next, nxt)
        drain(cur)

        def pair(p, c2):
            acc = jnp.zeros((16,), jnp.int32)
            for g in range(7):
                words = range(g * 3, min(g * 3 + 3, WRDS // 16))
                s = jnp.zeros((16,), jnp.int32)
                for k in words:
                    v = (rows_i[cur, p, pl.ds(k * 16, 16)]
                         & rows_j[cur, p, pl.ds(k * 16, 16)])
                    v = v - (lax.shift_right_logical(v, 1) & 0x55555555)
                    v = ((v & 0x33333333)
                         + (lax.shift_right_logical(v, 2) & 0x33333333))
                    s = s + v
                acc = acc + ((s + lax.shift_right_logical(s, 4))
                             & 0x1F1F1F1F)
            cnt_buf[p, :] = lax.shift_right_logical(
                acc * jnp.int32(0x01010101), 24)
            for k in range(IN_CH // 16):
                s = pl.ds(k * 16, 16)
                xij_buf[p, s] = xi[cur, p, s] * xj[cur, p, s]
            return c2
        lax.fori_loop(0, CH, pair, 0)
        pltpu.sync_copy(xij_buf, xij_hbm.at[pl.ds(off, CH)])
        pltpu.sync_copy(cnt_buf, counts_hbm.at[pl.ds(off, CH)])
        return carry

    lax.fori_loop(0, NCH, chunk, 0)
    drain(0)


def _sc_pairs(bm, tar0, tar1, tar0b, tar1b, x):
    mesh = plsc.VectorSubcoreMesh(core_axis_name="c", subcore_axis_name="s")
    f = pl.kernel(
        _pair_body,
        out_type=(jax.ShapeDtypeStruct((B, 16), jnp.int32),
                  jax.ShapeDtypeStruct((B, IN_CH), jnp.float32)),
        mesh=mesh,
        scratch_types=[
            pltpu.VMEM((2, CH), jnp.int32),
            pltpu.VMEM((2, CH), jnp.int32),
            pltpu.VMEM((2, CH), jnp.int32),
            pltpu.VMEM((2, CH), jnp.int32),
            pltpu.VMEM((2, CH, WRDS), jnp.int32),
            pltpu.VMEM((2, CH, WRDS), jnp.int32),
            pltpu.VMEM((2, CH, IN_CH), jnp.float32),
            pltpu.VMEM((2, CH, IN_CH), jnp.float32),
            pltpu.VMEM((CH, IN_CH), jnp.float32),
            pltpu.VMEM((CH, 16), jnp.int32),
            pltpu.SemaphoreType.DMA,
        ],
        compiler_params=pltpu.CompilerParams(use_tc_tiling_on_sc=False),
    )
    return f(bm, tar0, tar1, tar0b, tar1b, x)


def _table_body(w1, b1, w2, b2, w3, b3, beta, out_ref):
    c = lax.broadcasted_iota(jnp.int32, (TBL, 1), 0).astype(jnp.float32)
    h = jax.nn.relu(c * w1[...] + b1[...])
    h = jax.nn.relu(
        jax.lax.dot_general(h, w2[...], (((1,), (0,)), ((), ())),
                            preferred_element_type=jnp.float32) + b2[...])
    t = jax.lax.dot_general(h, w3[...], (((1,), (0,)), ((), ())),
                            preferred_element_type=jnp.float32) + b3[...]
    out_ref[...] = t * beta[...]


def _main_body(cnt_ref, xij_ref, table_ref, w1, b1, w2, b2, l1, lb1, l2, lb2,
               out_ref):
    xij = xij_ref[...]
    h = jax.nn.relu(
        jax.lax.dot_general(xij, w1[...], (((1,), (0,)), ((), ())),
                            preferred_element_type=jnp.float32) + b1[...])
    hij = jax.lax.dot_general(h, w2[...], (((1,), (0,)), ((), ())),
                              preferred_element_type=jnp.float32) + b2[...]
    c = jnp.sum(cnt_ref[...], axis=1, keepdims=True)
    onehot = (c == lax.broadcasted_iota(jnp.int32, (BLK, TBL), 1)
              ).astype(jnp.float32)
    hcn = jax.lax.dot_general(onehot, table_ref[...], (((1,), (0,)), ((), ())),
                              preferred_element_type=jnp.float32)
    z = hcn + hij
    h2 = jax.nn.relu(
        jax.lax.dot_general(z, l1[...], (((1,), (0,)), ((), ())),
                            preferred_element_type=jnp.float32) + lb1[...])
    out_ref[...] = jax.lax.dot_general(
        h2, l2[...], (((1,), (0,)), ((), ())),
        preferred_element_type=jnp.float32) + lb2[...]


EPAD = 320512
CE = 2048
ROWS_P = 1250
PLANE = ROWS_P * WRDS
DUMP = 4 * PLANE
CW = 2096
NCW = 12


def _build_body(srcp, dstp, out_hbm, spm, sbuf, dbuf, widx, wval, zbuf,
                pbuf, obuf, sem):
    cid = lax.axis_index("c")
    sid = lax.axis_index("s")

    def z16(i, c):
        zbuf[pl.ds(i * 16, 16)] = jnp.zeros((16,), jnp.int32)
        return c
    lax.fori_loop(0, 4000 // 16, z16, 0)

    def do_pass(p, carry):
        rlo = cid * 5000 + p * ROWS_P

        for g in range(5):
            hz = [pltpu.async_copy(
                zbuf,
                spm.at[pl.ds(sid * 100000 + (g * 5 + i) * 4000, 4000)], sem)
                for i in range(5)]
            for h in hz:
                h.wait()
        plsc.subcore_barrier()

        elim = sid * 20000 + 20000

        def chunk(ch, c):
            eoff = sid * 20000 + ch * CE
            pltpu.sync_copy(srcp.at[pl.ds(eoff, CE)], sbuf)
            pltpu.sync_copy(dstp.at[pl.ds(eoff, CE)], dbuf)

            hs = []
            for j in range(16):
                def vec(v8, c2):
                    v = j * 8 + v8
                    sl = pl.ds(v * 16, 16)
                    s = sbuf[sl]
                    d = dbuf[sl]
                    pos = eoff + v * 16 + lax.iota(jnp.int32, 16)
                    r = s - rlo
                    inr = (r >= 0) & (r < ROWS_P) & (pos < elim)
                    idx = (((d & 3) * ROWS_P + r) * WRDS
                           + lax.shift_right_logical(d, 5))
                    idx = jnp.where(inr, idx, DUMP + (pos & 127))
                    val = jnp.where(
                        inr,
                        jnp.int32(1)
                        << ((lax.shift_right_logical(d, 2) & 7) * 4),
                        0)
                    csl = pl.ds(v8 * 16, 16)
                    widx[j, csl] = idx
                    wval[j, csl] = val
                    return c2
                lax.fori_loop(0, 8, vec, 0)
                hs.append(pltpu.async_copy(
                    wval.at[j], spm.at[widx.at[j]], sem, add=True))
            for h in hs:
                h.wait()
            return c
        lax.fori_loop(0, 10, chunk, 0)
        plsc.subcore_barrier()

        def cchunk(k, c):
            poff = sid * 25000 + k * CW
            hp = [pltpu.async_copy(
                spm.at[pl.ds(q * PLANE + poff, CW)], pbuf.at[q], sem)
                for q in range(4)]
            for h in hp:
                h.wait()

            def cvec(v, c2):
                sl = pl.ds(v * 16, 16)
                o = jnp.zeros((16,), jnp.int32)
                for q in range(4):
                    w = pbuf[q, sl]
                    nz = ((w | lax.shift_right_logical(w, 1)
                           | lax.shift_right_logical(w, 2)
                           | lax.shift_right_logical(w, 3)) & 0x11111111)
                    o = o | (nz << q)
                obuf[sl] = o
                return c2
            lax.fori_loop(0, CW // 16, cvec, 0)
            pltpu.sync_copy(
                obuf, out_hbm.at[cid, pl.ds(p * PLANE + poff, CW)])
            return c
        lax.fori_loop(0, NCW, cchunk, 0)
        plsc.subcore_barrier()
        return carry

    lax.fori_loop(0, 4, do_pass, 0)


def _sc_build(edge_index):
    srcp = jnp.concatenate(
        [edge_index[0], jnp.zeros((EPAD - 320000,), jnp.int32)])
    dstp = jnp.concatenate(
        [edge_index[1], jnp.zeros((EPAD - 320000,), jnp.int32)])
    mesh = plsc.VectorSubcoreMesh(core_axis_name="c", subcore_axis_name="s")
    f = pl.kernel(
        _build_body,
        out_type=jax.ShapeDtypeStruct((2, 1600320), jnp.int32),
        mesh=mesh,
        scratch_types=[
            pltpu.VMEM_SHARED((4 * PLANE + 160,), jnp.int32),
            pltpu.VMEM((CE,), jnp.int32),
            pltpu.VMEM((CE,), jnp.int32),
            pltpu.VMEM((16, 128), jnp.int32),
            pltpu.VMEM((16, 128), jnp.int32),
            pltpu.VMEM((4000,), jnp.int32),
            pltpu.VMEM((4, CW), jnp.int32),
            pltpu.VMEM((CW,), jnp.int32),
            pltpu.SemaphoreType.DMA,
        ],
        compiler_params=pltpu.CompilerParams(use_tc_tiling_on_sc=False),
    )
    out = f(srcp, dstp)
    return out.reshape(10002, WRDS)


def kernel(x, edge_index, tar_ei, beta, xcn_w1, xcn_b1, xcn_w2, xcn_b2,
           xcn_w3, xcn_b3, xij_w1, xij_b1, xij_w2, xij_b2,
           lin_w1, lin_b1, lin_w2, lin_b2):
    bm = _sc_build(edge_index)
    tar0, tar1 = tar_ei[0], tar_ei[1]
    tar0b = tar0 + (tar0 >= 5000).astype(jnp.int32)
    tar1b = tar1 + (tar1 >= 5000).astype(jnp.int32)
    counts, xij = _sc_pairs(bm, tar0, tar1, tar0b, tar1b, x)

    full = lambda shape: pl.BlockSpec(shape, lambda *_: (0,) * len(shape))
    table = pl.pallas_call(
        _table_body,
        out_shape=jax.ShapeDtypeStruct((TBL, HID), jnp.float32),
        in_specs=[full((1, HID)), full((1, HID)), full((HID, HID)),
                  full((1, HID)), full((HID, HID)), full((1, HID)),
                  full((1, 1))],
        out_specs=full((TBL, HID)),
    )(xcn_w1, xcn_b1.reshape(1, HID), xcn_w2, xcn_b2.reshape(1, HID),
      xcn_w3, xcn_b3.reshape(1, HID), beta.reshape(1, 1))

    nb = B // BLK
    out = pl.pallas_call(
        _main_body,
        grid=(nb,),
        out_shape=jax.ShapeDtypeStruct((B, 1), jnp.float32),
        in_specs=[
            pl.BlockSpec((BLK, 16), lambda i: (i, 0)),
            pl.BlockSpec((BLK, IN_CH), lambda i: (i, 0)),
            full((TBL, HID)),
            full((IN_CH, HID)), full((1, HID)),
            full((HID, HID)), full((1, HID)),
            full((HID, HID)), full((1, HID)),
            full((HID, 1)), full((1, 1)),
        ],
        out_specs=pl.BlockSpec((BLK, 1), lambda i: (i, 0)),
    )(counts, xij, table,
      xij_w1, xij_b1.reshape(1, HID), xij_w2, xij_b2.reshape(1, HID),
      lin_w1, lin_b1.reshape(1, HID), lin_w2, lin_b2.reshape(1, 1))
    return out
